# baseline (speedup 1.0000x reference)
import jax
import jax.numpy as jnp
from jax.experimental import pallas as pl
from jax.experimental.pallas import tpu as pltpu

_HIDDEN = 1024
_EXPERTS = 64
_TOKENS = 32768
_BLK = 4096
_TBLK = 8192


def _matmul_block(h_ref, w_ref, logits_ref):
    logits_ref[...] = jnp.dot(
        h_ref[...], w_ref[...], preferred_element_type=jnp.float32
    )


def _top2_block(logits_ref, weights_ref, idx_ref):
    logits = logits_ref[...]
    ids_f = jax.lax.broadcasted_iota(jnp.int32, logits.shape, 1).astype(jnp.float32)
    m1 = jnp.max(logits, axis=1, keepdims=True)
    f1 = jnp.where(logits == m1, 1.0, 0.0)
    i1 = jnp.sum(f1 * ids_f, axis=1, keepdims=True)
    masked = jnp.where(f1 > 0.0, -jnp.inf, logits)
    m2 = jnp.max(masked, axis=1, keepdims=True)
    f2 = jnp.where(masked == m2, 1.0, 0.0)
    i2 = jnp.sum(f2 * ids_f, axis=1, keepdims=True)

    e = jnp.exp(m2 - m1)
    w1 = 1.0 / (1.0 + e)
    weights_ref[...] = jnp.concatenate([w1, 1.0 - w1], axis=1)
    idx_ref[...] = jnp.concatenate([i1, i2], axis=1).astype(jnp.int32)


def kernel(hidden_states, gate_weight):
    wt = gate_weight.T
    logits = pl.pallas_call(
        _matmul_block,
        grid=(_TOKENS // _BLK,),
        in_specs=[
            pl.BlockSpec((_BLK, _HIDDEN), lambda i: (i, 0)),
            pl.BlockSpec((_HIDDEN, _EXPERTS), lambda i: (0, 0)),
        ],
        out_specs=pl.BlockSpec((_BLK, _EXPERTS), lambda i: (i, 0)),
        out_shape=jax.ShapeDtypeStruct((_TOKENS, _EXPERTS), jnp.float32),
        compiler_params=pltpu.CompilerParams(
            dimension_semantics=("arbitrary",),
        ),
    )(hidden_states, wt)

    weights, idx = pl.pallas_call(
        _top2_block,
        grid=(_TOKENS // _TBLK,),
        in_specs=[pl.BlockSpec((_TBLK, _EXPERTS), lambda i: (i, 0))],
        out_specs=[
            pl.BlockSpec((_TBLK, 2), lambda i: (i, 0)),
            pl.BlockSpec((_TBLK, 2), lambda i: (i, 0)),
        ],
        out_shape=[
            jax.ShapeDtypeStruct((_TOKENS, 2), jnp.float32),
            jax.ShapeDtypeStruct((_TOKENS, 2), jnp.int32),
        ],
        compiler_params=pltpu.CompilerParams(
            dimension_semantics=("arbitrary",),
        ),
    )(logits)
    return (weights, idx, logits)
